# baseline (speedup 1.0000x reference)
import functools

import jax
import jax.numpy as jnp
import numpy as np
from jax.experimental import pallas as pl

H = 384
W = 384
N = H * W
D = 256
T = 200
TPAD = 256
NUM_CLASSES = 50
THRESHOLDS = (0.5, 1.0, 2.0)

BN = 16384
BIGF = 1e30
BIGI = 2 ** 30


def _fused_kernel(x_ref, t_ref, t2b_ref, code_ref, mask_ref, mind_ref, pred_ref):
    x = x_ref[...]
    t = t_ref[...]
    xt = jnp.dot(t, x, preferred_element_type=jnp.float32)
    s = t2b_ref[...] - 2.0 * xt
    smin = jnp.min(s, axis=0)
    x2 = jnp.sum(x * x, axis=0)
    mind = smin + x2
    sel = jnp.where(s == smin[None, :], code_ref[...], BIGI)
    code = jnp.min(sel, axis=0)
    pred_ref[...] = (code & 63)[None, :]
    mind_ref[...] = mind[None, :]
    mask_ref[...] = jnp.concatenate(
        [(mind[None, :] <= th).astype(jnp.int8) for th in THRESHOLDS], axis=0)


@functools.partial(jax.jit, static_argnames=())
def kernel(frame_embeddings, templates, template_classes):
    x = frame_embeddings.reshape(D, N)
    t = jnp.zeros((TPAD, D), jnp.float32).at[:T, :].set(templates)
    t2 = jnp.sum(t * t, axis=1, keepdims=True)
    bias = jnp.where(
        jnp.arange(TPAD, dtype=jnp.int32)[:, None] < T, 0.0, BIGF)
    t2b = t2 + bias
    iota = jnp.arange(TPAD, dtype=jnp.int32)[:, None]
    code = jnp.where(
        iota < T,
        iota * 64 + jnp.pad(template_classes, (0, TPAD - T))[:, None],
        BIGI)

    nb = N // BN
    mask8, mind, pred = pl.pallas_call(
        _fused_kernel,
        grid=(nb,),
        in_specs=[
            pl.BlockSpec((D, BN), lambda i: (0, i)),
            pl.BlockSpec((TPAD, D), lambda i: (0, 0)),
            pl.BlockSpec((TPAD, 1), lambda i: (0, 0)),
            pl.BlockSpec((TPAD, 1), lambda i: (0, 0)),
        ],
        out_specs=[
            pl.BlockSpec((3, BN), lambda i: (0, i)),
            pl.BlockSpec((1, BN), lambda i: (0, i)),
            pl.BlockSpec((1, BN), lambda i: (0, i)),
        ],
        out_shape=[
            jax.ShapeDtypeStruct((3, N), jnp.int8),
            jax.ShapeDtypeStruct((1, N), jnp.float32),
            jax.ShapeDtypeStruct((1, N), jnp.int32),
        ],
    )(x, t, t2b, code)

    mask_list = mask8.astype(jnp.bool_).reshape(3, 1, N)
    return mask_list, mind, pred


# speedup vs baseline: 1.1015x; 1.1015x over previous
import functools

import jax
import jax.numpy as jnp
import numpy as np
from jax.experimental import pallas as pl

H = 384
W = 384
N = H * W
D = 256
T = 200
TPAD = 256
NUM_CLASSES = 50
THRESHOLDS = (0.5, 1.0, 2.0)

BN = 8192
BIGF = 1e30
BIGI = 2 ** 30


def _fused_kernel(x_ref, tm_ref, t2b_ref, code_ref, ones_ref,
                  mask_ref, mind_ref, pred_ref):
    x = x_ref[...]
    xb = x.astype(jnp.bfloat16)
    xtm = jnp.dot(tm_ref[...], xb, preferred_element_type=jnp.float32)
    s = t2b_ref[...] + xtm
    smin = jnp.min(s, axis=0)
    xsq = xb * xb
    x2 = jnp.dot(ones_ref[...], xsq,
                 preferred_element_type=jnp.float32)[0]
    mind = smin + x2
    sel = jnp.where(s == smin[None, :], code_ref[...], BIGF)
    code = jnp.min(sel, axis=0).astype(jnp.int32)
    pred_ref[...] = (code & 63)[None, :]
    mind_ref[...] = mind[None, :]
    mask_ref[...] = jnp.concatenate(
        [(mind[None, :] <= th).astype(jnp.int8) for th in THRESHOLDS], axis=0)


@functools.partial(jax.jit, static_argnames=())
def kernel(frame_embeddings, templates, template_classes):
    x = frame_embeddings.reshape(D, N)
    t = jnp.zeros((TPAD, D), jnp.float32).at[:T, :].set(templates)
    tm = (-2.0 * t).astype(jnp.bfloat16)
    t2 = jnp.sum(t * t, axis=1, keepdims=True)
    bias = jnp.where(
        jnp.arange(TPAD, dtype=jnp.int32)[:, None] < T, 0.0, BIGF)
    t2b = t2 + bias
    iota = jnp.arange(TPAD, dtype=jnp.int32)[:, None]
    code = jnp.where(
        iota < T,
        iota * 64 + jnp.pad(template_classes, (0, TPAD - T))[:, None],
        BIGI).astype(jnp.float32)
    ones8 = jnp.zeros((8, D), jnp.bfloat16).at[0, :].set(1.0)

    nb = N // BN
    mask8, mind, pred = pl.pallas_call(
        _fused_kernel,
        grid=(nb,),
        in_specs=[
            pl.BlockSpec((D, BN), lambda i: (0, i)),
            pl.BlockSpec((TPAD, D), lambda i: (0, 0)),
            pl.BlockSpec((TPAD, 1), lambda i: (0, 0)),
            pl.BlockSpec((TPAD, 1), lambda i: (0, 0)),
            pl.BlockSpec((8, D), lambda i: (0, 0)),
        ],
        out_specs=[
            pl.BlockSpec((3, BN), lambda i: (0, i)),
            pl.BlockSpec((1, BN), lambda i: (0, i)),
            pl.BlockSpec((1, BN), lambda i: (0, i)),
        ],
        out_shape=[
            jax.ShapeDtypeStruct((3, N), jnp.int8),
            jax.ShapeDtypeStruct((1, N), jnp.float32),
            jax.ShapeDtypeStruct((1, N), jnp.int32),
        ],
    )(x, tm, t2b, code, ones8)

    mask_list = mask8.astype(jnp.bool_).reshape(3, 1, N)
    return mask_list, mind, pred
